# baseline (speedup 1.0000x reference)
import functools

import jax
import jax.numpy as jnp
from jax import lax
from jax.experimental import pallas as pl
from jax.experimental.pallas import tpu as pltpu
from jax.experimental.pallas import tpu_sc as plsc

ROWS = 10000
COLS = 10000
B = 16384
NC, NS = 2, 16
NW = NC * NS
BPW = B // NW
L = 16
GRP = 32
NGRP = BPW // GRP
NCHUNK = BPW // 128


def _body(idx_hbm, table_hbm, out_hbm, r_v, c_v, m_v, r8_v, rows_v, out_v, win_sh, sem, sem2):
    cid = lax.axis_index("c")
    sid = lax.axis_index("s")
    wid = sid * NC + cid
    base = wid * BPW
    pltpu.sync_copy(idx_hbm.at[pl.ds(base, BPW)], r_v)
    pltpu.sync_copy(idx_hbm.at[pl.ds(B + base, BPW)], c_v)

    lane_ids = jax.lax.iota(jnp.int32, L)
    for v in range(BPW // L):
        rv = r_v[pl.ds(v * L, L)]
        for k in range(L // 2):
            p = v * (L // 2) + k
            lo = jnp.full((L,), rv[2 * k], jnp.int32)
            hi = jnp.full((L,), rv[2 * k + 1], jnp.int32)
            r8_v[pl.ds(p * L, L)] = jnp.where(lane_ids < 8, lo, hi)

    H = BPW // 2

    def _mk_group(half_sem, half):
        def _group(g, carry):
            g0 = pl.multiple_of(g * GRP, GRP)
            cvecs = [c_v[pl.ds(g0 + v * L, L)] for v in range(GRP // L)]
            for u in range(GRP):
                e = g * GRP + u
                c0 = pl.multiple_of((cvecs[u // L][u % L] >> 7) * 128, 128)
                e8 = pl.multiple_of(e * 8, 8)
                pltpu.async_copy(
                    table_hbm.at[r8_v.at[pl.ds(e8, 1)], pl.ds(c0, 128)],
                    rows_v.at[pl.ds(e, 1), :],
                    half_sem,
                )
            return carry

        lax.fori_loop(half * (NGRP // 2), (half + 1) * (NGRP // 2), _group, 0)

    _mk_group(sem, 0)
    _mk_group(sem2, 1)

    win_lin = win_sh.at[0]
    for h, hsem in ((0, sem), (1, sem2)):
        pltpu.make_async_copy(
            table_hbm.at[pl.ds(0, H), pl.ds(0, 128)],
            rows_v.at[pl.ds(h * H, H), :],
            hsem,
        ).wait()
        pltpu.sync_copy(
            rows_v.at[pl.ds(h * H, H), :], win_sh.at[pl.ds(sid * H, H), :]
        )
        for i in range(H // L):
            s = pl.ds(h * H + i * L, L)
            m_v[s] = (sid * H + jax.lax.iota(jnp.int32, L) + i * L) * 128 + (
                c_v[s] & 127
            )
        copies = []
        for k in range(H // 128):
            s = pl.ds(h * H + k * 128, 128)
            copies.append(pltpu.async_copy(win_lin.at[m_v.at[s]], out_v.at[s], hsem))
        for cp in copies:
            cp.wait()
    pltpu.sync_copy(out_v, out_hbm.at[pl.ds(base, BPW)])


_table_lookup = functools.partial(
    pl.kernel,
    mesh=plsc.VectorSubcoreMesh(core_axis_name="c", subcore_axis_name="s"),
    out_type=jax.ShapeDtypeStruct((B,), jnp.float32),
    scratch_types=[
        pltpu.VMEM((BPW,), jnp.int32),
        pltpu.VMEM((BPW,), jnp.int32),
        pltpu.VMEM((BPW,), jnp.int32),
        pltpu.VMEM((BPW * 8,), jnp.int32),
        pltpu.VMEM((BPW, 128), jnp.float32),
        pltpu.VMEM((BPW,), jnp.float32),
        pltpu.VMEM_SHARED((NS * (BPW // 2), 128), jnp.float32),
        pltpu.SemaphoreType.DMA,
        pltpu.SemaphoreType.DMA,
    ],
)(_body)


def kernel(indices, table):
    idx_flat = indices.astype(jnp.int32).reshape(-1)
    return _table_lookup(idx_flat, table)


# speedup vs baseline: 1.0283x; 1.0283x over previous
import functools

import jax
import jax.numpy as jnp
from jax import lax
from jax.experimental import pallas as pl
from jax.experimental.pallas import tpu as pltpu
from jax.experimental.pallas import tpu_sc as plsc

ROWS = 10000
COLS = 10000
B = 16384
NC, NS = 2, 16
NW = NC * NS
BPW = B // NW
L = 16
GRP = 16
NGRP = BPW // GRP
NCHUNK = BPW // 128


def _body(idx_hbm, table_hbm, out_hbm, r_v, c_v, m_v, r8_v, rows_v, out_v, win_sh, sem, sem2):
    cid = lax.axis_index("c")
    sid = lax.axis_index("s")
    wid = sid * NC + cid
    base = wid * BPW
    pltpu.sync_copy(idx_hbm.at[pl.ds(base, BPW)], r_v)
    pltpu.sync_copy(idx_hbm.at[pl.ds(B + base, BPW)], c_v)

    lane_ids = jax.lax.iota(jnp.int32, L)
    for v in range(BPW // L):
        rv = r_v[pl.ds(v * L, L)]
        for k in range(L // 2):
            p = v * (L // 2) + k
            lo = jnp.full((L,), rv[2 * k], jnp.int32)
            hi = jnp.full((L,), rv[2 * k + 1], jnp.int32)
            r8_v[pl.ds(p * L, L)] = jnp.where(lane_ids < 8, lo, hi)

    H = BPW // 2

    def _mk_group(half_sem, half):
        def _group(g, carry):
            cvec = c_v[pl.ds(pl.multiple_of(g * GRP, GRP), GRP)]
            for u in range(GRP):
                e = g * GRP + u
                c0 = pl.multiple_of((cvec[u] >> 7) * 128, 128)
                e8 = pl.multiple_of(e * 8, 8)
                pltpu.async_copy(
                    table_hbm.at[r8_v.at[pl.ds(e8, 1)], pl.ds(c0, 128)],
                    rows_v.at[pl.ds(e, 1), :],
                    half_sem,
                )
            return carry

        lax.fori_loop(half * (NGRP // 2), (half + 1) * (NGRP // 2), _group, 0)

    _mk_group(sem, 0)
    _mk_group(sem2, 1)

    win_lin = win_sh.at[0]
    for h, hsem in ((0, sem), (1, sem2)):
        pltpu.make_async_copy(
            table_hbm.at[pl.ds(0, H), pl.ds(0, 128)],
            rows_v.at[pl.ds(h * H, H), :],
            hsem,
        ).wait()
        pltpu.sync_copy(
            rows_v.at[pl.ds(h * H, H), :], win_sh.at[pl.ds(sid * H, H), :]
        )
        for i in range(H // L):
            s = pl.ds(h * H + i * L, L)
            m_v[s] = (sid * H + jax.lax.iota(jnp.int32, L) + i * L) * 128 + (
                c_v[s] & 127
            )
        copies = []
        for k in range(H // 128):
            s = pl.ds(h * H + k * 128, 128)
            copies.append(pltpu.async_copy(win_lin.at[m_v.at[s]], out_v.at[s], hsem))
        for cp in copies:
            cp.wait()
    pltpu.sync_copy(out_v, out_hbm.at[pl.ds(base, BPW)])


_table_lookup = functools.partial(
    pl.kernel,
    mesh=plsc.VectorSubcoreMesh(core_axis_name="c", subcore_axis_name="s"),
    out_type=jax.ShapeDtypeStruct((B,), jnp.float32),
    scratch_types=[
        pltpu.VMEM((BPW,), jnp.int32),
        pltpu.VMEM((BPW,), jnp.int32),
        pltpu.VMEM((BPW,), jnp.int32),
        pltpu.VMEM((BPW * 8,), jnp.int32),
        pltpu.VMEM((BPW, 128), jnp.float32),
        pltpu.VMEM((BPW,), jnp.float32),
        pltpu.VMEM_SHARED((NS * (BPW // 2), 128), jnp.float32),
        pltpu.SemaphoreType.DMA,
        pltpu.SemaphoreType.DMA,
    ],
)(_body)


def kernel(indices, table):
    idx_flat = indices.astype(jnp.int32).reshape(-1)
    return _table_lookup(idx_flat, table)


# speedup vs baseline: 1.0286x; 1.0004x over previous
import functools

import jax
import jax.numpy as jnp
from jax import lax
from jax.experimental import pallas as pl
from jax.experimental.pallas import tpu as pltpu
from jax.experimental.pallas import tpu_sc as plsc

ROWS = 10000
COLS = 10000
B = 16384
NC, NS = 2, 16
NW = NC * NS
BPW = B // NW
L = 16
GRP = 16
NGRP = BPW // GRP
NCHUNK = BPW // 128


def _body(idx_hbm, table_hbm, out_hbm, r_v, c_v, m_v, r8_v, rows_v, out_v, win_sh, sem, sem2):
    cid = lax.axis_index("c")
    sid = lax.axis_index("s")
    wid = sid * NC + cid
    base = wid * BPW
    pltpu.sync_copy(idx_hbm.at[pl.ds(base, BPW)], r_v)
    pltpu.sync_copy(idx_hbm.at[pl.ds(B + base, BPW)], c_v)

    lane_ids = jax.lax.iota(jnp.int32, L)

    def _replicate(half):
        for v in range(half * (BPW // L // 2), (half + 1) * (BPW // L // 2)):
            rv = r_v[pl.ds(v * L, L)]
            for k in range(L // 2):
                p = v * (L // 2) + k
                lo = jnp.full((L,), rv[2 * k], jnp.int32)
                hi = jnp.full((L,), rv[2 * k + 1], jnp.int32)
                r8_v[pl.ds(p * L, L)] = jnp.where(lane_ids < 8, lo, hi)

    H = BPW // 2

    def _mk_group(half_sem, half):
        def _group(g, carry):
            cvec = c_v[pl.ds(pl.multiple_of(g * GRP, GRP), GRP)]
            for u in range(GRP):
                e = g * GRP + u
                c0 = pl.multiple_of((cvec[u] >> 7) * 128, 128)
                e8 = pl.multiple_of(e * 8, 8)
                pltpu.async_copy(
                    table_hbm.at[r8_v.at[pl.ds(e8, 1)], pl.ds(c0, 128)],
                    rows_v.at[pl.ds(e, 1), :],
                    half_sem,
                )
            return carry

        lax.fori_loop(half * (NGRP // 2), (half + 1) * (NGRP // 2), _group, 0)

    _replicate(0)
    _mk_group(sem, 0)
    _replicate(1)
    _mk_group(sem2, 1)

    win_lin = win_sh.at[0]
    for h, hsem in ((0, sem), (1, sem2)):
        pltpu.make_async_copy(
            table_hbm.at[pl.ds(0, H), pl.ds(0, 128)],
            rows_v.at[pl.ds(h * H, H), :],
            hsem,
        ).wait()
        pltpu.sync_copy(
            rows_v.at[pl.ds(h * H, H), :], win_sh.at[pl.ds(sid * H, H), :]
        )
        for i in range(H // L):
            s = pl.ds(h * H + i * L, L)
            m_v[s] = (sid * H + jax.lax.iota(jnp.int32, L) + i * L) * 128 + (
                c_v[s] & 127
            )
        copies = []
        for k in range(H // 128):
            s = pl.ds(h * H + k * 128, 128)
            copies.append(pltpu.async_copy(win_lin.at[m_v.at[s]], out_v.at[s], hsem))
        for cp in copies:
            cp.wait()
    pltpu.sync_copy(out_v, out_hbm.at[pl.ds(base, BPW)])


_table_lookup = functools.partial(
    pl.kernel,
    mesh=plsc.VectorSubcoreMesh(core_axis_name="c", subcore_axis_name="s"),
    out_type=jax.ShapeDtypeStruct((B,), jnp.float32),
    scratch_types=[
        pltpu.VMEM((BPW,), jnp.int32),
        pltpu.VMEM((BPW,), jnp.int32),
        pltpu.VMEM((BPW,), jnp.int32),
        pltpu.VMEM((BPW * 8,), jnp.int32),
        pltpu.VMEM((BPW, 128), jnp.float32),
        pltpu.VMEM((BPW,), jnp.float32),
        pltpu.VMEM_SHARED((NS * (BPW // 2), 128), jnp.float32),
        pltpu.SemaphoreType.DMA,
        pltpu.SemaphoreType.DMA,
    ],
)(_body)


def kernel(indices, table):
    idx_flat = indices.astype(jnp.int32).reshape(-1)
    return _table_lookup(idx_flat, table)
